# baseline (speedup 1.0000x reference)
import math

import jax
import jax.numpy as jnp
import numpy as np
from jax import lax
from jax.experimental import pallas as pl
from jax.experimental.pallas import tpu as pltpu
from jax.experimental.pallas import tpu_sc as plsc

N_POLY = 4
SEGMENTS = 100000
OUT_F = 64
OUT_P = 128
BATCH = 4096

NC, NS, L = 2, 16, 16
NW = NC * NS
B_PER_W = BATCH // NW
N_CHUNKS = B_PER_W // L
NSLOT = 4
PREF = 2
TPC = 256
SH = 8

_NODES = (-np.cos(np.arange(N_POLY) * math.pi / (N_POLY - 1))).astype(np.float32)
_DENS = []
for _j in range(N_POLY):
    _d = np.float32(1.0)
    for _m in range(N_POLY):
        if _m != _j:
            _d = np.float32(_d * np.float32(_NODES[_j] - _NODES[_m]))
    _DENS.append(float(_d))
_N0, _N1, _N2, _N3 = (float(v) for v in _NODES)

_BIG = np.int32(2**30)


def _body(xs_hbm, perm_hbm, w_hbm, out_hbm, xv, pvv, sv, k2v, basisv, tiles,
          wout, semr, sem1):
    wid = lax.axis_index("s") * NC + lax.axis_index("c")
    base = wid * B_PER_W

    pltpu.sync_copy(xs_hbm.at[pl.ds(base, B_PER_W)], xv)
    pltpu.sync_copy(perm_hbm.at[pl.ds(base, B_PER_W)], pvv)

    iota = lax.iota(jnp.int32, L)

    def idx_chunk(c, _):
        xc = xv[pl.ds(c * L, L)]
        t = (xc + 1.0) / 2.0 * float(SEGMENTS)
        iid = t.astype(jnp.int32)
        idf = iid.astype(jnp.float32)
        s = iid * 3
        sv[pl.ds(c * L, L)] = s
        k2v[pl.ds(c * L, L)] = jnp.right_shift(s + (N_POLY - 1), SH)
        x_min = idf / float(SEGMENTS) * 2.0 - 1.0
        x_max = (idf + 1.0) / float(SEGMENTS) * 2.0 - 1.0
        xi = 2.0 * ((xc - x_min) / (x_max - x_min)) - 1.0
        b0 = (xi - _N1) * (xi - _N2) * (xi - _N3) / _DENS[0]
        b1 = (xi - _N0) * (xi - _N2) * (xi - _N3) / _DENS[1]
        b2 = (xi - _N0) * (xi - _N1) * (xi - _N3) / _DENS[2]
        b3 = (xi - _N0) * (xi - _N1) * (xi - _N2) / _DENS[3]
        basisv[pl.ds(0 * B_PER_W + c * L, L)] = b0
        basisv[pl.ds(1 * B_PER_W + c * L, L)] = b1
        basisv[pl.ds(2 * B_PER_W + c * L, L)] = b2
        basisv[pl.ds(3 * B_PER_W + c * L, L)] = b3
        return 0

    lax.fori_loop(0, N_CHUNKS, idx_chunk, 0)
    k2v[pl.ds(B_PER_W, L)] = jnp.full((L,), _BIG, dtype=jnp.int32)

    first = sv[pl.ds(0, L)]
    last = sv[pl.ds(B_PER_W - L, L)]
    lo = jnp.right_shift(first[0], SH)
    hi = jnp.right_shift(last[L - 1] + (N_POLY - 1), SH)
    n_t = hi - lo + 1

    def fire(j):
        off = pl.multiple_of((lo + j) * TPC, 128)
        pltpu.async_copy(
            w_hbm.at[:, pl.ds(off, TPC)],
            tiles.at[jnp.bitwise_and(j, NSLOT - 1)],
            semr.at[jnp.bitwise_and(j, NSLOT - 1)],
        )

    for _j in range(PREF):
        @pl.when(jnp.int32(_j) < n_t)
        def _(_j=_j):
            fire(jnp.int32(_j))

    def process(ptr, t):
        ispl = jnp.full((L,), ptr, dtype=jnp.int32)
        s_spl = plsc.load_gather(sv, [ispl])
        bn = [
            plsc.load_gather(basisv, [ispl + (n * B_PER_W)])
            for n in range(N_POLY)
        ]
        slots = []
        cols = []
        for n in range(N_POLY):
            tn = s_spl + n
            j = jnp.right_shift(tn, SH) - lo
            slots.append(jnp.bitwise_and(j, NSLOT - 1))
            cols.append(jnp.bitwise_and(tn, TPC - 1))
        for q in range(OUT_F // L):
            oq = q * L + iota
            acc = None
            for n in range(N_POLY):
                wv = plsc.load_gather(tiles, [slots[n], oq, cols[n]])
                acc = wv * bn[n] if acc is None else acc + wv * bn[n]
            wout[ptr, pl.ds(q * L, L)] = acc

    def tile_step(t, carry):
        ptr, k2cur = carry

        @pl.when(t + PREF < n_t)
        def _():
            fire(t + PREF)

        slot = jnp.bitwise_and(t, NSLOT - 1)
        pltpu.make_async_copy(
            w_hbm.at[:, pl.ds(0, TPC)], tiles.at[slot], semr.at[slot]
        ).wait()

        def run_cond(c):
            p, k2 = c
            return k2 == lo + t

        def run_body(c):
            p, _ = c
            process(p, t)
            pn = p + 1
            k2n = plsc.load_gather(k2v, [jnp.full((L,), pn, dtype=jnp.int32)])
            return pn, k2n[0]

        ptr, k2cur = lax.while_loop(run_cond, run_body, (ptr, k2cur))
        return ptr, k2cur

    k20 = plsc.load_gather(k2v, [jnp.zeros((L,), dtype=jnp.int32)])
    lax.fori_loop(0, n_t, tile_step, (jnp.int32(0), k20[0]))

    pltpu.async_copy(wout, out_hbm.at[pvv], sem1).wait()


@jax.jit
def kernel(x, w):
    xs, perm = lax.sort(
        (x.reshape(BATCH), lax.iota(jnp.int32, BATCH)), num_keys=1
    )
    mesh = plsc.VectorSubcoreMesh(
        core_axis_name="c", subcore_axis_name="s", num_cores=NC, num_subcores=NS
    )
    out_pad = pl.kernel(
        _body,
        out_type=jax.ShapeDtypeStruct((BATCH, OUT_P), jnp.float32),
        mesh=mesh,
        compiler_params=pltpu.CompilerParams(
            use_tc_tiling_on_sc=True,
            needs_layout_passes=False,
            disable_bounds_checks=True,
        ),
        scratch_types=[
            pltpu.VMEM((B_PER_W,), jnp.float32),
            pltpu.VMEM((B_PER_W,), jnp.int32),
            pltpu.VMEM((B_PER_W + L,), jnp.int32),
            pltpu.VMEM((B_PER_W + L,), jnp.int32),
            pltpu.VMEM((N_POLY * B_PER_W,), jnp.float32),
            pltpu.VMEM((NSLOT, OUT_F, TPC), jnp.float32),
            pltpu.VMEM((B_PER_W, OUT_P), jnp.float32),
            pltpu.SemaphoreType.DMA((NSLOT,)),
            pltpu.SemaphoreType.DMA,
        ],
    )(xs, perm, w)
    return out_pad[:, :OUT_F]


# speedup vs baseline: 1.0169x; 1.0169x over previous
import math

import jax
import jax.numpy as jnp
import numpy as np
from jax import lax
from jax.experimental import pallas as pl
from jax.experimental.pallas import tpu as pltpu
from jax.experimental.pallas import tpu_sc as plsc

N_POLY = 4
SEGMENTS = 100000
OUT_F = 64
OUT_P = 128
BATCH = 4096

NC, NS, L = 2, 16, 16
NW = NC * NS
B_PER_W = BATCH // NW
N_CHUNKS = B_PER_W // L
NSLOT = 8
PREF = 6

_NODES = (-np.cos(np.arange(N_POLY) * math.pi / (N_POLY - 1))).astype(np.float32)
_DENS = []
for _j in range(N_POLY):
    _d = np.float32(1.0)
    for _m in range(N_POLY):
        if _m != _j:
            _d = np.float32(_d * np.float32(_NODES[_j] - _NODES[_m]))
    _DENS.append(float(_d))
_N0, _N1, _N2, _N3 = (float(v) for v in _NODES)

_BIG = np.int32(2**30)


def _body(xs_hbm, perm_hbm, w_hbm, out_hbm, xv, pvv, sv, k2v, basisv, tiles,
          wout, semr, sem1):
    wid = lax.axis_index("s") * NC + lax.axis_index("c")
    base = wid * B_PER_W

    pltpu.sync_copy(xs_hbm.at[pl.ds(base, B_PER_W)], xv)
    pltpu.sync_copy(perm_hbm.at[pl.ds(base, B_PER_W)], pvv)

    iota = lax.iota(jnp.int32, L)

    def idx_chunk(c, _):
        xc = xv[pl.ds(c * L, L)]
        t = (xc + 1.0) / 2.0 * float(SEGMENTS)
        iid = t.astype(jnp.int32)
        idf = iid.astype(jnp.float32)
        s = iid * 3
        sv[pl.ds(c * L, L)] = s
        k2v[pl.ds(c * L, L)] = jnp.right_shift(s + (N_POLY - 1), 7)
        x_min = idf / float(SEGMENTS) * 2.0 - 1.0
        x_max = (idf + 1.0) / float(SEGMENTS) * 2.0 - 1.0
        xi = 2.0 * ((xc - x_min) / (x_max - x_min)) - 1.0
        b0 = (xi - _N1) * (xi - _N2) * (xi - _N3) / _DENS[0]
        b1 = (xi - _N0) * (xi - _N2) * (xi - _N3) / _DENS[1]
        b2 = (xi - _N0) * (xi - _N1) * (xi - _N3) / _DENS[2]
        b3 = (xi - _N0) * (xi - _N1) * (xi - _N2) / _DENS[3]
        basisv[pl.ds(0 * B_PER_W + c * L, L)] = b0
        basisv[pl.ds(1 * B_PER_W + c * L, L)] = b1
        basisv[pl.ds(2 * B_PER_W + c * L, L)] = b2
        basisv[pl.ds(3 * B_PER_W + c * L, L)] = b3
        return 0

    lax.fori_loop(0, N_CHUNKS, idx_chunk, 0)
    k2v[pl.ds(B_PER_W, L)] = jnp.full((L,), _BIG, dtype=jnp.int32)

    first = sv[pl.ds(0, L)]
    last = sv[pl.ds(B_PER_W - L, L)]
    lo = jnp.right_shift(first[0], 7)
    hi = jnp.right_shift(last[L - 1] + (N_POLY - 1), 7)
    n_t = hi - lo + 1

    def fire(j):
        off = pl.multiple_of((lo + j) * 128, 128)
        pltpu.async_copy(
            w_hbm.at[:, pl.ds(off, 128)],
            tiles.at[jnp.bitwise_and(j, NSLOT - 1)],
            semr.at[jnp.bitwise_and(j, NSLOT - 1)],
        )

    for _j in range(PREF):
        @pl.when(jnp.int32(_j) < n_t)
        def _(_j=_j):
            fire(jnp.int32(_j))

    def process(ptr, t):
        ispl = jnp.full((L,), ptr, dtype=jnp.int32)
        s_spl = plsc.load_gather(sv, [ispl])
        bn = [
            plsc.load_gather(basisv, [ispl + (n * B_PER_W)])
            for n in range(N_POLY)
        ]
        slots = []
        cols = []
        for n in range(N_POLY):
            tn = s_spl + n
            j = jnp.right_shift(tn, 7) - lo
            slots.append(jnp.bitwise_and(j, NSLOT - 1))
            cols.append(jnp.bitwise_and(tn, 127))
        for q in range(OUT_F // L):
            oq = q * L + iota
            acc = None
            for n in range(N_POLY):
                wv = plsc.load_gather(tiles, [slots[n], oq, cols[n]])
                acc = wv * bn[n] if acc is None else acc + wv * bn[n]
            wout[ptr, pl.ds(q * L, L)] = acc

    def tile_step(t, carry):
        ptr, k2cur = carry

        @pl.when(t + PREF < n_t)
        def _():
            fire(t + PREF)

        slot = jnp.bitwise_and(t, NSLOT - 1)
        pltpu.make_async_copy(
            w_hbm.at[:, pl.ds(0, 128)], tiles.at[slot], semr.at[slot]
        ).wait()

        def run_cond(c):
            p, k2 = c
            return k2 == lo + t

        def run_body(c):
            p, _ = c
            process(p, t)
            pn = p + 1
            k2n = plsc.load_gather(k2v, [jnp.full((L,), pn, dtype=jnp.int32)])
            return pn, k2n[0]

        ptr, k2cur = lax.while_loop(run_cond, run_body, (ptr, k2cur))
        return ptr, k2cur

    k20 = plsc.load_gather(k2v, [jnp.zeros((L,), dtype=jnp.int32)])
    lax.fori_loop(0, n_t, tile_step, (jnp.int32(0), k20[0]))

    pltpu.async_copy(wout, out_hbm.at[pvv], sem1).wait()


@jax.jit
def kernel(x, w):
    xs, perm = lax.sort(
        (x.reshape(BATCH), lax.iota(jnp.int32, BATCH)), num_keys=1
    )
    mesh = plsc.VectorSubcoreMesh(
        core_axis_name="c", subcore_axis_name="s", num_cores=NC, num_subcores=NS
    )
    out_pad = pl.kernel(
        _body,
        out_type=jax.ShapeDtypeStruct((BATCH, OUT_P), jnp.float32),
        mesh=mesh,
        compiler_params=pltpu.CompilerParams(
            use_tc_tiling_on_sc=True,
            needs_layout_passes=False,
            disable_bounds_checks=True,
        ),
        scratch_types=[
            pltpu.VMEM((B_PER_W,), jnp.float32),
            pltpu.VMEM((B_PER_W,), jnp.int32),
            pltpu.VMEM((B_PER_W + L,), jnp.int32),
            pltpu.VMEM((B_PER_W + L,), jnp.int32),
            pltpu.VMEM((N_POLY * B_PER_W,), jnp.float32),
            pltpu.VMEM((NSLOT, OUT_F, 128), jnp.float32),
            pltpu.VMEM((B_PER_W, OUT_P), jnp.float32),
            pltpu.SemaphoreType.DMA((NSLOT,)),
            pltpu.SemaphoreType.DMA,
        ],
    )(xs, perm, w)
    return out_pad[:, :OUT_F]
